# baseline (speedup 1.0000x reference)
import functools

import jax
import jax.numpy as jnp
from jax import lax
from jax.experimental import pallas as pl
from jax.experimental.pallas import tpu as pltpu
from jax.experimental.pallas import tpu_sc as plsc

_N = 4096
_R = 256
_K = 10


def _bitonic_topk_plan(n, k):
    ex = []
    kk = 2
    while kk <= n:
        j = kk // 2
        while j >= 1:
            for i in range(n):
                l = i ^ j
                if l > i:
                    ex.append((i, l, (i & kk) == 0))
            j //= 2
        kk *= 2
    needed = set(range(k))
    plan = []
    for (i, l, d) in reversed(ex):
        ni, nl = i in needed, l in needed
        if not (ni or nl):
            continue
        plan.append((i, l, d, ni, nl))
        needed.add(i)
        needed.add(l)
    plan.reverse()
    return plan


_PLAN16 = _bitonic_topk_plan(16, _K)


def _lane_topk(chunks, plan):
    a = list(chunks)
    for (i, l, desc, ni, nl) in plan:
        x, y = a[i], a[l]
        hi = jnp.maximum(x, y) if (ni if desc else nl) else None
        lo = jnp.minimum(x, y) if (nl if desc else ni) else None
        if desc:
            if ni:
                a[i] = hi
            if nl:
                a[l] = lo
        else:
            if ni:
                a[i] = lo
            if nl:
                a[l] = hi
    return a


def _candidates(s):
    r, n = s.shape
    nchunks = n // 128
    chunks = [s[:, g * 128:(g + 1) * 128] for g in range(nchunks)]
    a = _lane_topk(chunks[:16], _PLAN16)
    b = _lane_topk(chunks[16:], _PLAN16)
    return jnp.concatenate(
        [jnp.maximum(a[i], b[_K - 1 - i]) for i in range(_K)], axis=1)


def _pass_a(wr_ref, wc_ref, s_ref, c_ref):
    i = pl.program_id(0)
    wr = wr_ref[...]
    wc = wc_ref[...]
    r, n = wr.shape
    s = jnp.maximum(jnp.maximum(wr, 0.0), jnp.maximum(wc, 0.0).T)
    col = jax.lax.broadcasted_iota(jnp.int32, (r, n), 1)
    row = jax.lax.broadcasted_iota(jnp.int32, (r, n), 0) + i * r
    s = jnp.where(col == row, s + 1.0, s)
    s_ref[...] = s
    c_ref[...] = _candidates(s)


def _pass_b(s_ref, t_ref, o_ref):
    s = s_ref[...]
    t = t_ref[0, 0, :][:, None]
    masked = jnp.where(s >= t, s, 0.0)
    ssum = jnp.sum(masked, axis=1, keepdims=True)
    o_ref[...] = masked * (1.0 / (ssum + 1e-8))


def _lane_reduce(x, op, scratch):
    del scratch
    acc = x[0]
    for j in range(1, 16):
        acc = op(acc, x[j])
    return acc


def _sc_thresholds(cands):
    n, w = cands.shape
    info = plsc.get_sparse_core_info()
    nw = info.num_cores * info.num_subcores
    rows_w = n // nw
    rb = 8
    unroll = 8
    mesh = plsc.VectorSubcoreMesh(core_axis_name="c", subcore_axis_name="s")

    @functools.partial(
        pl.kernel, mesh=mesh,
        out_type=jax.ShapeDtypeStruct((n, 16), jnp.float32),
        scratch_types=[
            pltpu.VMEM((rb, w), jnp.float32),
            pltpu.VMEM((rows_w, 16), jnp.float32),
            pltpu.VMEM((16,), jnp.float32),
        ],
    )
    def k(c_hbm, t_hbm, buf, tbuf, sbuf):
        wid = lax.axis_index("s") * info.num_cores + lax.axis_index("c")
        base = wid * rows_w

        def batch_body(bi, _):
            pltpu.sync_copy(c_hbm.at[pl.ds(base + bi * rb, rb)], buf)

            def row_body(rr, _):
                def one_iter(_k2, carry):
                    t, c = carry
                    t16 = jnp.full((16,), t, jnp.float32)

                    def scan_m(j, macc):
                        for u in range(unroll):
                            x = buf[rr, pl.ds((j * unroll + u) * 16, 16)]
                            macc = jnp.maximum(
                                macc, jnp.where(x < t16, x, -1.0))
                        return macc

                    macc = lax.fori_loop(
                        0, w // 16 // unroll, scan_m,
                        jnp.full((16,), -1.0, jnp.float32))
                    m = _lane_reduce(macc, jnp.maximum, sbuf)
                    m16 = jnp.full((16,), m, jnp.float32)

                    def scan_c(j, cacc):
                        for u in range(unroll):
                            x = buf[rr, pl.ds((j * unroll + u) * 16, 16)]
                            cacc = cacc + jnp.where(x >= m16, 1.0, 0.0)
                        return cacc

                    cacc = lax.fori_loop(
                        0, w // 16 // unroll, scan_c,
                        jnp.zeros((16,), jnp.float32))
                    cnt = _lane_reduce(cacc, jnp.add, sbuf)
                    upd = c < float(_K)
                    return (jnp.where(upd, m, t), jnp.where(upd, cnt, c))

                t, _c = lax.fori_loop(
                    0, _K, one_iter,
                    (jnp.float32(jnp.inf), jnp.float32(0.0)))
                tbuf[bi * rb + rr] = jnp.full((16,), t, jnp.float32)
                return 0

            lax.fori_loop(0, rb, row_body, 0)
            return 0

        lax.fori_loop(0, rows_w // rb, batch_body, 0)
        pltpu.sync_copy(tbuf, t_hbm.at[pl.ds(base, rows_w)])

    return k(cands)


def kernel(W):
    n = W.shape[0]
    g = n // _R
    row_spec = pl.BlockSpec((_R, n), lambda i: (i, 0))
    col_spec = pl.BlockSpec((n, _R), lambda i: (0, i))
    cand_spec = pl.BlockSpec((_R, 1280), lambda i: (i, 0))
    vec_spec = pl.BlockSpec((1, 1, _R), lambda i: (i, 0, 0))

    s_full, cands = pl.pallas_call(
        _pass_a,
        grid=(g,),
        in_specs=[row_spec, col_spec],
        out_specs=[row_spec, cand_spec],
        out_shape=[
            jax.ShapeDtypeStruct((n, n), jnp.float32),
            jax.ShapeDtypeStruct((n, 1280), jnp.float32),
        ],
    )(W, W)

    t = _sc_thresholds(cands)[:, 0].reshape(g, 1, _R)

    return pl.pallas_call(
        _pass_b,
        grid=(g,),
        in_specs=[row_spec, vec_spec],
        out_specs=row_spec,
        out_shape=jax.ShapeDtypeStruct((n, n), jnp.float32),
    )(s_full, t)


# speedup vs baseline: 4.3543x; 4.3543x over previous
import jax
import jax.numpy as jnp
from jax.experimental import pallas as pl

_N = 4096
_R = 256
_K = 10


def _bitonic_topk_plan(n, k):
    ex = []
    kk = 2
    while kk <= n:
        j = kk // 2
        while j >= 1:
            for i in range(n):
                l = i ^ j
                if l > i:
                    ex.append((i, l, (i & kk) == 0))
            j //= 2
        kk *= 2
    needed = set(range(k))
    plan = []
    for (i, l, d) in reversed(ex):
        ni, nl = i in needed, l in needed
        if not (ni or nl):
            continue
        plan.append((i, l, d, ni, nl))
        needed.add(i)
        needed.add(l)
    plan.reverse()
    return plan


_PLAN16 = _bitonic_topk_plan(16, _K)


def _lane_topk(chunks, plan):
    a = list(chunks)
    for (i, l, desc, ni, nl) in plan:
        x, y = a[i], a[l]
        hi = jnp.maximum(x, y) if (ni if desc else nl) else None
        lo = jnp.minimum(x, y) if (nl if desc else ni) else None
        if desc:
            if ni:
                a[i] = hi
            if nl:
                a[l] = lo
        else:
            if ni:
                a[i] = lo
            if nl:
                a[l] = hi
    return a


def _threshold(s):
    r, n = s.shape
    nchunks = n // 128
    if nchunks == 32:
        chunks = [s[:, g * 128:(g + 1) * 128] for g in range(nchunks)]
        a = _lane_topk(chunks[:16], _PLAN16)
        b = _lane_topk(chunks[16:], _PLAN16)
        cands = jnp.concatenate(
            [jnp.maximum(a[i], b[_K - 1 - i]) for i in range(_K)], axis=1)
    else:
        cands = s

    t = jnp.max(cands, axis=1, keepdims=True)
    c = jnp.sum(jnp.where(cands >= t, 1.0, 0.0), axis=1, keepdims=True)
    for _ in range(_K - 1):
        m = jnp.max(jnp.where(cands < t, cands, -1.0), axis=1, keepdims=True)
        cnt = jnp.sum(jnp.where(cands >= m, 1.0, 0.0), axis=1, keepdims=True)
        upd = c < float(_K)
        t = jnp.where(upd, m, t)
        c = jnp.where(upd, cnt, c)
    return t


def _fused(wr_ref, wc_ref, o_ref):
    i = pl.program_id(0)
    wr = wr_ref[...]
    wc = wc_ref[...]
    r, n = wr.shape
    s = jnp.maximum(jnp.maximum(wr, 0.0), jnp.maximum(wc, 0.0).T)
    col = jax.lax.broadcasted_iota(jnp.int32, (r, n), 1)
    row = jax.lax.broadcasted_iota(jnp.int32, (r, n), 0) + i * r
    s = jnp.where(col == row, s + 1.0, s)
    t = _threshold(s)
    masked = jnp.where(s >= t, s, 0.0)
    ssum = jnp.sum(masked, axis=1, keepdims=True)
    o_ref[...] = masked * (1.0 / (ssum + 1e-8))


def kernel(W):
    n = W.shape[0]
    g = n // _R
    row_spec = pl.BlockSpec((_R, n), lambda i: (i, 0))
    col_spec = pl.BlockSpec((n, _R), lambda i: (0, i))
    return pl.pallas_call(
        _fused,
        grid=(g,),
        in_specs=[row_spec, col_spec],
        out_specs=row_spec,
        out_shape=jax.ShapeDtypeStruct((n, n), jnp.float32),
    )(W, W)
